# baseline (speedup 1.0000x reference)
import functools

import jax
import jax.numpy as jnp
from jax import lax
from jax.experimental import pallas as pl
from jax.experimental.pallas import tpu as pltpu
from jax.experimental.pallas import tpu_sc as plsc

_LB = 128
_CT = 8


@functools.lru_cache(maxsize=None)
def _make_gather(b0, b1, dim):
    info = plsc.get_sparse_core_info()
    nc, ns = info.num_cores, info.num_subcores
    nw = nc * ns
    n_bt = b0 // _LB
    n_ct = dim // _CT
    assert n_bt == nw
    mesh = plsc.VectorSubcoreMesh(core_axis_name="c", subcore_axis_name="s")

    @functools.partial(
        pl.kernel,
        mesh=mesh,
        out_type=jax.ShapeDtypeStruct((b1, n_ct, n_bt, _CT, _LB), jnp.float32),
        scratch_types=[
            pltpu.VMEM((b1, _LB), jnp.int32),
            pltpu.VMEM((2, _LB, dim), jnp.float32),
            pltpu.VMEM((2, dim, _LB), jnp.float32),
            pltpu.SemaphoreType.DMA((2,)),
            pltpu.SemaphoreType.DMA((2,)),
            pltpu.SemaphoreType.DMA,
        ],
        compiler_params=pltpu.CompilerParams(
            use_tc_tiling_on_sc=False, needs_layout_passes=False
        ),
    )
    def gather_kernel(table_hbm, tt_hbm, out_hbm, idx_v, rows_v, trans_v,
                      gsem, osem, isem):
        j = lax.axis_index("s") * nc + lax.axis_index("c")
        pltpu.async_copy(
            tt_hbm.at[pl.ds(0, b1), pl.ds(j * _LB, _LB)], idx_v, isem
        ).wait()
        ci = lax.iota(jnp.int32, 16)

        def gather_copy(s, p):
            return pltpu.make_async_copy(
                table_hbm.at[idx_v.at[s]], rows_v.at[p], gsem.at[p]
            )

        def out_copies(s, p):
            return [
                pltpu.make_async_copy(
                    trans_v.at[p, pl.ds(k * _CT, _CT)],
                    out_hbm.at[s, k, j],
                    osem.at[p],
                )
                for k in range(n_ct)
            ]

        cvecs = [ci + (fg * 16) for fg in range(dim // 16)]

        def transpose(p):
            def tbody(g, carry):
                b0 = g * 8
                gv = jnp.zeros((16,), jnp.int32) + b0
                for db in range(8):
                    b = b0 + db
                    bv = gv + db
                    for fg in range(dim // 16):
                        vec = rows_v[p, b, pl.ds(fg * 16, 16)]
                        plsc.store_scatter(trans_v.at[p], [cvecs[fg], bv], vec)
                return carry
            lax.fori_loop(0, _LB // 8, tbody, 0)

        def step(s, p, first, issue_next=True):
            if issue_next:
                gather_copy(s + 1, 1 - p).start()
            gather_copy(s, p).wait()
            if not first:
                for c in out_copies(s - 2, p):
                    c.wait()
            transpose(p)
            for c in out_copies(s, p):
                c.start()

        gather_copy(0, 0).start()

        def pair(i, carry):
            s0 = 2 * i + 2
            step(s0, 0, False)
            step(s0 + 1, 1, False)
            return carry

        step(0, 0, True)
        step(1, 1, True)
        lax.fori_loop(0, (b1 - 4) // 2, pair, 0)
        step(b1 - 2, 0, False)
        step(b1 - 1, 1, False, issue_next=False)
        for c in out_copies(b1 - 2, 0):
            c.wait()
        for c in out_copies(b1 - 1, 1):
            c.wait()

    return gather_kernel


def kernel(token_ids, embedding_matrix):
    b0, b1 = token_ids.shape
    _, d = embedding_matrix.shape
    tt = token_ids.astype(jnp.int32).T
    out5d = _make_gather(b0, b1, d)(embedding_matrix, tt)
    return out5d.transpose(2, 4, 0, 1, 3).reshape(b0, b1, d)


# speedup vs baseline: 1.1051x; 1.1051x over previous
import functools

import jax
import jax.numpy as jnp
from jax import lax
from jax.experimental import pallas as pl
from jax.experimental.pallas import tpu as pltpu
from jax.experimental.pallas import tpu_sc as plsc

_SEQ_STEP = 16
_NSLOT = 2


@functools.lru_cache(maxsize=None)
def _make_gather(b0, b1, dim):
    info = plsc.get_sparse_core_info()
    nc, ns = info.num_cores, info.num_subcores
    nw = nc * ns
    seq_per_w = b0 // nw
    n_steps = seq_per_w // _SEQ_STEP
    step_rows = _SEQ_STEP * b1
    b_per_w = seq_per_w * b1
    mesh = plsc.VectorSubcoreMesh(core_axis_name="c", subcore_axis_name="s")

    @functools.partial(
        pl.kernel,
        mesh=mesh,
        out_type=jax.ShapeDtypeStruct((b0, b1, dim), jnp.float32),
        scratch_types=[
            pltpu.VMEM((seq_per_w, b1), jnp.int32),
            pltpu.VMEM((_NSLOT, _SEQ_STEP, b1, dim), jnp.float32),
            pltpu.SemaphoreType.DMA((_NSLOT,)),
            pltpu.SemaphoreType.DMA((_NSLOT,)),
            pltpu.SemaphoreType.DMA,
        ],
        compiler_params=pltpu.CompilerParams(use_tc_tiling_on_sc=False),
    )
    def gather_kernel(table_hbm, idx_hbm, out_hbm, idx_v, rows_v, gsem, osem, isem):
        wid = lax.axis_index("s") * nc + lax.axis_index("c")
        seq0 = wid * seq_per_w
        pltpu.async_copy(idx_hbm.at[pl.ds(seq0, seq_per_w)], idx_v, isem).wait()

        def seq_gather(s, j):
            b = s % _NSLOT
            return pltpu.make_async_copy(
                table_hbm.at[idx_v.at[s * _SEQ_STEP + j]],
                rows_v.at[b, j],
                gsem.at[b],
            )

        def gather_start(s):
            for j in range(_SEQ_STEP):
                seq_gather(s, j).start()

        def gather_wait(s):
            for j in range(_SEQ_STEP):
                seq_gather(s, j).wait()

        def out_copy(s):
            b = s % _NSLOT
            return pltpu.make_async_copy(
                rows_v.at[b],
                out_hbm.at[pl.ds(seq0 + s * _SEQ_STEP, _SEQ_STEP)],
                osem.at[b],
            )

        for s in range(n_steps):
            if s >= _NSLOT:
                out_copy(s - _NSLOT).wait()
            gather_start(s)
            if s >= 1:
                gather_wait(s - 1)
                out_copy(s - 1).start()
        gather_wait(n_steps - 1)
        out_copy(n_steps - 1).start()
        for s in range(max(n_steps - _NSLOT, 0), n_steps):
            out_copy(s).wait()

    return gather_kernel


def kernel(token_ids, embedding_matrix):
    b0, b1 = token_ids.shape
    _, d = embedding_matrix.shape
    return _make_gather(b0, b1, d)(embedding_matrix, token_ids.astype(jnp.int32))
